# baseline (speedup 1.0000x reference)
import functools

import jax
import jax.numpy as jnp
from jax import lax
from jax.experimental import pallas as pl
from jax.experimental.pallas import tpu as pltpu, tpu_sc as plsc

N_SITES = 50000
N_IN = 128
N_OUT = 128
HALF = N_OUT // 2
FV = 9

SCALE = 1000.0
BIAS_Q = 3640
SUM_BIAS = FV * BIAS_Q

NW = 32
CHUNK = 1568
SUB = 784
STRIP = 392
CHUNK_L = N_SITES - (NW - 1) * CHUNK
SUB_L = CHUNK_L // 2
STRIP_L = SUB_L // 3
BN = 1024


def _mm_body(feat_ref, w_ref, b_ref, *out_refs):
    x = feat_ref[...]
    t_all = (
        jnp.dot(x, w_ref[...], preferred_element_type=jnp.float32)
        + b_ref[0]
    )
    q_all = (t_all * SCALE + (BIAS_Q + 0.5)).astype(jnp.int32)
    for k in range(FV):
        q = q_all[:, k * N_OUT:(k + 1) * N_OUT]
        out_refs[k][...] = q[:, :HALF] | (q[:, HALF:] << 16)


def _transform(features, w_cat, b_cat):
    grid = (pl.cdiv(N_SITES, BN),)
    return pl.pallas_call(
        _mm_body,
        grid=grid,
        in_specs=[
            pl.BlockSpec((BN, N_IN), lambda i: (i, 0)),
            pl.BlockSpec((N_IN, FV * N_OUT), lambda i: (0, 0)),
            pl.BlockSpec((1, FV * N_OUT), lambda i: (0, 0)),
        ],
        out_specs=[
            pl.BlockSpec((BN, HALF), lambda i: (i, 0)) for _ in range(FV)
        ],
        out_shape=[
            jax.ShapeDtypeStruct((N_SITES, HALF), jnp.int32)
            for _ in range(FV)
        ],
    )(features, w_cat, b_cat)


def _work(t_refs, idx_hbm, out_hbm, idx_v, acc_v, stg, sem,
          base, chunk, sub, strip):
    for f in range(FV):
        pltpu.sync_copy(
            idx_hbm.at[pl.ds(f * N_SITES + base, chunk)],
            idx_v.at[pl.ds(f * chunk, chunk)],
        )
    for i in range(chunk // sub):
        off = base + i * sub
        acc = acc_v.at[pl.ds(0, sub)]
        pltpu.async_copy(
            t_refs[0].at[idx_v.at[pl.ds(i * sub, sub)]], acc, sem
        ).wait()
        for f in range(1, FV):
            pltpu.async_copy(
                t_refs[f].at[idx_v.at[pl.ds(f * chunk + i * sub, sub)]],
                acc,
                sem,
                add=True,
            ).wait()
        for j in range(sub // strip):
            roff = j * strip

            def row(r, _):
                for w in range(HALF // 16):
                    word = acc_v[roff + r, pl.ds(w * 16, 16)]
                    lo = (word & 0xFFFF) - SUM_BIAS
                    hi = lax.shift_right_logical(word, 16) - SUM_BIAS
                    stg[r, pl.ds(w * 16, 16)] = (
                        lo.astype(jnp.float32) * (1.0 / SCALE)
                    )
                    stg[r, pl.ds(HALF + w * 16, 16)] = (
                        hi.astype(jnp.float32) * (1.0 / SCALE)
                    )
                return 0

            lax.fori_loop(0, strip, row, 0)
            pltpu.sync_copy(
                stg.at[pl.ds(0, strip)],
                out_hbm.at[pl.ds(off + roff, strip)],
            )


def _sc_body(*refs):
    t_refs = refs[:FV]
    idx_hbm, out_hbm, idx_v, acc_v, stg, sem = refs[FV:]
    c = lax.axis_index("c")
    s = lax.axis_index("s")
    wid = s * 2 + c
    base = wid * CHUNK

    @pl.when(wid < NW - 1)
    def _full():
        _work(t_refs, idx_hbm, out_hbm, idx_v, acc_v, stg, sem,
              base, CHUNK, SUB, STRIP)

    @pl.when(wid == NW - 1)
    def _last():
        _work(t_refs, idx_hbm, out_hbm, idx_v, acc_v, stg, sem,
              base, CHUNK_L, SUB_L, STRIP_L)


_gather_sum = functools.partial(
    pl.kernel,
    out_type=jax.ShapeDtypeStruct((N_SITES, N_OUT), jnp.float32),
    mesh=plsc.VectorSubcoreMesh(core_axis_name="c", subcore_axis_name="s"),
    compiler_params=pltpu.CompilerParams(use_tc_tiling_on_sc=False),
    scratch_types=[
        pltpu.VMEM((FV * CHUNK,), jnp.int32),
        pltpu.VMEM((SUB, HALF), jnp.int32),
        pltpu.VMEM((STRIP, N_OUT), jnp.float32),
        pltpu.SemaphoreType.DMA,
    ],
)(_sc_body)


@jax.jit
def kernel(features, neighbor_idx, weight, bias):
    w_cat = weight.transpose(1, 0, 2).reshape(N_IN, FV * N_OUT)
    b_cat = jnp.tile(bias * (1.0 / FV), (FV,)).reshape(1, FV * N_OUT)
    t_list = _transform(features, w_cat, b_cat)
    idx_flat = neighbor_idx.T.reshape(FV * N_SITES)
    return _gather_sum(*t_list, idx_flat)


# speedup vs baseline: 1.9522x; 1.9522x over previous
import functools

import jax
import jax.numpy as jnp
from jax import lax
from jax.experimental import pallas as pl
from jax.experimental.pallas import tpu as pltpu, tpu_sc as plsc

N_SITES = 50000
N_IN = 128
N_OUT = 128
FV = 9

NW = 32
CHUNK = 1568
SUB = 784
CHUNK_L = N_SITES - (NW - 1) * CHUNK
SUB_L = CHUNK_L // 2
BN = 1024


def _mm_body(feat_ref, w_ref, b_ref, out_ref):
    x = feat_ref[...]
    t_all = (
        jnp.dot(x, w_ref[...], preferred_element_type=jnp.float32)
        + b_ref[0]
    )
    for k in range(FV):
        out_ref[k] = t_all[:, k * N_OUT:(k + 1) * N_OUT]


def _transform(features, w_cat, b_cat):
    grid = (pl.cdiv(N_SITES, BN),)
    return pl.pallas_call(
        _mm_body,
        grid=grid,
        in_specs=[
            pl.BlockSpec((BN, N_IN), lambda i: (i, 0)),
            pl.BlockSpec((N_IN, FV * N_OUT), lambda i: (0, 0)),
            pl.BlockSpec((1, FV * N_OUT), lambda i: (0, 0)),
        ],
        out_specs=pl.BlockSpec((FV, BN, N_OUT), lambda i: (0, i, 0)),
        out_shape=jax.ShapeDtypeStruct((FV, N_SITES, N_OUT), jnp.float32),
    )(features, w_cat, b_cat)


def _work(t_hbm, idx_hbm, out_hbm, idx_v, acc_v, sem, base, chunk, sub):
    for f in range(FV):
        pltpu.sync_copy(
            idx_hbm.at[pl.ds(f * N_SITES + base, chunk)],
            idx_v.at[pl.ds(f * chunk, chunk)],
        )
    for i in range(chunk // sub):
        off = base + i * sub
        acc = acc_v.at[pl.ds(0, sub)]
        pltpu.async_copy(
            t_hbm.at[idx_v.at[pl.ds(i * sub, sub)]], acc, sem
        ).wait()
        for f in range(1, FV):
            pltpu.async_copy(
                t_hbm.at[idx_v.at[pl.ds(f * chunk + i * sub, sub)]],
                acc,
                sem,
                add=True,
            ).wait()
        pltpu.sync_copy(acc, out_hbm.at[pl.ds(off, sub)])


def _sc_body(t_hbm, idx_hbm, out_hbm, idx_v, acc_v, sem):
    c = lax.axis_index("c")
    s = lax.axis_index("s")
    wid = s * 2 + c
    base = wid * CHUNK

    @pl.when(wid < NW - 1)
    def _full():
        _work(t_hbm, idx_hbm, out_hbm, idx_v, acc_v, sem, base, CHUNK, SUB)

    @pl.when(wid == NW - 1)
    def _last():
        _work(t_hbm, idx_hbm, out_hbm, idx_v, acc_v, sem, base, CHUNK_L, SUB_L)


_gather_sum = functools.partial(
    pl.kernel,
    out_type=jax.ShapeDtypeStruct((N_SITES, N_OUT), jnp.float32),
    mesh=plsc.VectorSubcoreMesh(core_axis_name="c", subcore_axis_name="s"),
    scratch_types=[
        pltpu.VMEM((FV * CHUNK,), jnp.int32),
        pltpu.VMEM((SUB, N_OUT), jnp.float32),
        pltpu.SemaphoreType.DMA,
    ],
)(_sc_body)


@jax.jit
def kernel(features, neighbor_idx, weight, bias):
    w_cat = weight.transpose(1, 0, 2).reshape(N_IN, FV * N_OUT)
    b_cat = jnp.tile(bias * (1.0 / FV), (FV,)).reshape(1, FV * N_OUT)
    t = _transform(features, w_cat, b_cat)
    t_flat = t.reshape(FV * N_SITES, N_OUT)
    idx_t = (
        neighbor_idx.T
        + (jnp.arange(FV, dtype=jnp.int32) * N_SITES)[:, None]
    )
    return _gather_sum(t_flat, idx_t.reshape(FV * N_SITES))


# speedup vs baseline: 1.9766x; 1.0125x over previous
import functools

import jax
import jax.numpy as jnp
from jax import lax
from jax.experimental import pallas as pl
from jax.experimental.pallas import tpu as pltpu, tpu_sc as plsc

N_SITES = 50000
N_IN = 128
N_OUT = 128
FV = 9

NW = 32
CHUNK = 1568
SUB = 784
CHUNK_L = N_SITES - (NW - 1) * CHUNK
SUB_L = CHUNK_L // 2
BN = 1024


def _mm_body(feat_ref, w_ref, b_ref, out_ref):
    x = feat_ref[...]
    for k in range(FV):
        out_ref[k] = (
            jnp.dot(x, w_ref[k], preferred_element_type=jnp.float32)
            + b_ref[0] * (1.0 / FV)
        )


def _transform(features, weight, bias):
    grid = (pl.cdiv(N_SITES, BN),)
    return pl.pallas_call(
        _mm_body,
        grid=grid,
        in_specs=[
            pl.BlockSpec((BN, N_IN), lambda i: (i, 0)),
            pl.BlockSpec((FV, N_IN, N_OUT), lambda i: (0, 0, 0)),
            pl.BlockSpec((1, N_OUT), lambda i: (0, 0)),
        ],
        out_specs=pl.BlockSpec((FV, BN, N_OUT), lambda i: (0, i, 0)),
        out_shape=jax.ShapeDtypeStruct((FV, N_SITES, N_OUT), jnp.float32),
    )(features, weight, bias.reshape(1, N_OUT))


def _work(t_hbm, idx_hbm, out_hbm, idx_v, acc_v, sem, base, chunk, sub):
    for f in range(FV):
        pltpu.sync_copy(
            idx_hbm.at[pl.ds(f * N_SITES + base, chunk)],
            idx_v.at[pl.ds(f * chunk, chunk)],
        )
    for i in range(chunk // sub):
        off = base + i * sub
        acc = acc_v.at[pl.ds(0, sub)]
        pltpu.async_copy(
            t_hbm.at[idx_v.at[pl.ds(i * sub, sub)]], acc, sem
        ).wait()
        for f in range(1, FV):
            pltpu.async_copy(
                t_hbm.at[idx_v.at[pl.ds(f * chunk + i * sub, sub)]],
                acc,
                sem,
                add=True,
            ).wait()
        pltpu.sync_copy(acc, out_hbm.at[pl.ds(off, sub)])


def _sc_body(t_hbm, idx_hbm, out_hbm, idx_v, acc_v, sem):
    c = lax.axis_index("c")
    s = lax.axis_index("s")
    wid = s * 2 + c
    base = wid * CHUNK

    @pl.when(wid < NW - 1)
    def _full():
        _work(t_hbm, idx_hbm, out_hbm, idx_v, acc_v, sem, base, CHUNK, SUB)

    @pl.when(wid == NW - 1)
    def _last():
        _work(t_hbm, idx_hbm, out_hbm, idx_v, acc_v, sem, base, CHUNK_L, SUB_L)


_gather_sum = functools.partial(
    pl.kernel,
    out_type=jax.ShapeDtypeStruct((N_SITES, N_OUT), jnp.float32),
    mesh=plsc.VectorSubcoreMesh(core_axis_name="c", subcore_axis_name="s"),
    scratch_types=[
        pltpu.VMEM((FV * CHUNK,), jnp.int32),
        pltpu.VMEM((SUB, N_OUT), jnp.float32),
        pltpu.SemaphoreType.DMA,
    ],
)(_sc_body)


@jax.jit
def kernel(features, neighbor_idx, weight, bias):
    t = _transform(features, weight, bias)
    t_flat = t.reshape(FV * N_SITES, N_OUT)
    idx_t = (
        neighbor_idx.T
        + (jnp.arange(FV, dtype=jnp.int32) * N_SITES)[:, None]
    )
    return _gather_sum(t_flat, idx_t.reshape(FV * N_SITES))


# speedup vs baseline: 2.0647x; 1.0446x over previous
import functools

import jax
import jax.numpy as jnp
from jax import lax
from jax.experimental import pallas as pl
from jax.experimental.pallas import tpu as pltpu, tpu_sc as plsc

N_SITES = 50000
N_IN = 128
N_OUT = 128
FV = 9

NW = 32
CHUNK = 1568
SUB = 784
CHUNK_L = N_SITES - (NW - 1) * CHUNK
SUB_L = CHUNK_L // 2
BN = 2048


def _mm_body(feat_ref, w_ref, b_ref, out_ref):
    x = feat_ref[...]
    for k in range(FV):
        out_ref[k] = (
            jnp.dot(x, w_ref[k], preferred_element_type=jnp.float32)
            + b_ref[0] * (1.0 / FV)
        )


def _transform(features, weight, bias):
    grid = (pl.cdiv(N_SITES, BN),)
    return pl.pallas_call(
        _mm_body,
        grid=grid,
        in_specs=[
            pl.BlockSpec((BN, N_IN), lambda i: (i, 0)),
            pl.BlockSpec((FV, N_IN, N_OUT), lambda i: (0, 0, 0)),
            pl.BlockSpec((1, N_OUT), lambda i: (0, 0)),
        ],
        out_specs=pl.BlockSpec((FV, BN, N_OUT), lambda i: (0, i, 0)),
        out_shape=jax.ShapeDtypeStruct((FV, N_SITES, N_OUT), jnp.float32),
    )(features, weight, bias.reshape(1, N_OUT))


def _work(t_hbm, idx_hbm, out_hbm, idx_v, acc_v, sem, base, chunk, sub):
    for f in range(FV):
        pltpu.sync_copy(
            idx_hbm.at[pl.ds(f * N_SITES + base, chunk)],
            idx_v.at[pl.ds(f * chunk, chunk)],
        )
    for i in range(chunk // sub):
        off = base + i * sub
        acc = acc_v.at[pl.ds(0, sub)]
        pltpu.async_copy(
            t_hbm.at[idx_v.at[pl.ds(i * sub, sub)]], acc, sem
        ).wait()
        for f in range(1, FV):
            pltpu.async_copy(
                t_hbm.at[idx_v.at[pl.ds(f * chunk + i * sub, sub)]],
                acc,
                sem,
                add=True,
            ).wait()
        pltpu.sync_copy(acc, out_hbm.at[pl.ds(off, sub)])


def _sc_body(t_hbm, idx_hbm, out_hbm, idx_v, acc_v, sem):
    c = lax.axis_index("c")
    s = lax.axis_index("s")
    wid = s * 2 + c
    base = wid * CHUNK

    @pl.when(wid < NW - 1)
    def _full():
        _work(t_hbm, idx_hbm, out_hbm, idx_v, acc_v, sem, base, CHUNK, SUB)

    @pl.when(wid == NW - 1)
    def _last():
        _work(t_hbm, idx_hbm, out_hbm, idx_v, acc_v, sem, base, CHUNK_L, SUB_L)


_gather_sum = functools.partial(
    pl.kernel,
    out_type=jax.ShapeDtypeStruct((N_SITES, N_OUT), jnp.float32),
    mesh=plsc.VectorSubcoreMesh(core_axis_name="c", subcore_axis_name="s"),
    scratch_types=[
        pltpu.VMEM((FV * CHUNK,), jnp.int32),
        pltpu.VMEM((SUB, N_OUT), jnp.float32),
        pltpu.SemaphoreType.DMA,
    ],
)(_sc_body)


@jax.jit
def kernel(features, neighbor_idx, weight, bias):
    t = _transform(features, weight, bias)
    t_flat = t.reshape(FV * N_SITES, N_OUT)
    idx_t = (
        neighbor_idx.T
        + (jnp.arange(FV, dtype=jnp.int32) * N_SITES)[:, None]
    )
    return _gather_sum(t_flat, idx_t.reshape(FV * N_SITES))


# speedup vs baseline: 2.0696x; 1.0024x over previous
import functools

import jax
import jax.numpy as jnp
from jax import lax
from jax.experimental import pallas as pl
from jax.experimental.pallas import tpu as pltpu, tpu_sc as plsc

N_SITES = 50000
N_IN = 128
N_OUT = 128
FV = 9

NW = 32
CHUNK = 1568
SUB = 784
CHUNK_L = N_SITES - (NW - 1) * CHUNK
SUB_L = CHUNK_L // 2
BN = 4096


def _mm_body(feat_ref, w_ref, b_ref, out_ref):
    x = feat_ref[...]
    for k in range(FV):
        out_ref[k] = (
            jnp.dot(x, w_ref[k], preferred_element_type=jnp.float32)
            + b_ref[0] * (1.0 / FV)
        )


def _transform(features, weight, bias):
    grid = (pl.cdiv(N_SITES, BN),)
    return pl.pallas_call(
        _mm_body,
        grid=grid,
        in_specs=[
            pl.BlockSpec((BN, N_IN), lambda i: (i, 0)),
            pl.BlockSpec((FV, N_IN, N_OUT), lambda i: (0, 0, 0)),
            pl.BlockSpec((1, N_OUT), lambda i: (0, 0)),
        ],
        out_specs=pl.BlockSpec((FV, BN, N_OUT), lambda i: (0, i, 0)),
        out_shape=jax.ShapeDtypeStruct((FV, N_SITES, N_OUT), jnp.float32),
    )(features, weight, bias.reshape(1, N_OUT))


def _work(t_hbm, idx_hbm, out_hbm, idx_v, acc_v, sem, base, chunk, sub):
    for f in range(FV):
        pltpu.sync_copy(
            idx_hbm.at[pl.ds(f * N_SITES + base, chunk)],
            idx_v.at[pl.ds(f * chunk, chunk)],
        )
    for i in range(chunk // sub):
        off = base + i * sub
        acc = acc_v.at[pl.ds(0, sub)]
        pltpu.async_copy(
            t_hbm.at[idx_v.at[pl.ds(i * sub, sub)]], acc, sem
        ).wait()
        for f in range(1, FV):
            pltpu.async_copy(
                t_hbm.at[idx_v.at[pl.ds(f * chunk + i * sub, sub)]],
                acc,
                sem,
                add=True,
            ).wait()
        pltpu.sync_copy(acc, out_hbm.at[pl.ds(off, sub)])


def _sc_body(t_hbm, idx_hbm, out_hbm, idx_v, acc_v, sem):
    c = lax.axis_index("c")
    s = lax.axis_index("s")
    wid = s * 2 + c
    base = wid * CHUNK

    @pl.when(wid < NW - 1)
    def _full():
        _work(t_hbm, idx_hbm, out_hbm, idx_v, acc_v, sem, base, CHUNK, SUB)

    @pl.when(wid == NW - 1)
    def _last():
        _work(t_hbm, idx_hbm, out_hbm, idx_v, acc_v, sem, base, CHUNK_L, SUB_L)


_gather_sum = functools.partial(
    pl.kernel,
    out_type=jax.ShapeDtypeStruct((N_SITES, N_OUT), jnp.float32),
    mesh=plsc.VectorSubcoreMesh(core_axis_name="c", subcore_axis_name="s"),
    scratch_types=[
        pltpu.VMEM((FV * CHUNK,), jnp.int32),
        pltpu.VMEM((SUB, N_OUT), jnp.float32),
        pltpu.SemaphoreType.DMA,
    ],
)(_sc_body)


@jax.jit
def kernel(features, neighbor_idx, weight, bias):
    t = _transform(features, weight, bias)
    t_flat = t.reshape(FV * N_SITES, N_OUT)
    idx_t = (
        neighbor_idx.T
        + (jnp.arange(FV, dtype=jnp.int32) * N_SITES)[:, None]
    )
    return _gather_sum(t_flat, idx_t.reshape(FV * N_SITES))


# speedup vs baseline: 2.0703x; 1.0003x over previous
import functools

import jax
import jax.numpy as jnp
from jax import lax
from jax.experimental import pallas as pl
from jax.experimental.pallas import tpu as pltpu, tpu_sc as plsc

N_SITES = 50000
N_IN = 128
N_OUT = 128
FV = 9

NW = 32
CHUNK = 1568
SUB = 784
CHUNK_L = N_SITES - (NW - 1) * CHUNK
SUB_L = CHUNK_L // 2
BN = 4096


def _mm_body(feat_ref, w_ref, b_ref, out_ref):
    x = feat_ref[...]
    for k in range(FV):
        out_ref[k] = (
            jnp.dot(x, w_ref[k], preferred_element_type=jnp.float32)
            + b_ref[0] * (1.0 / FV)
        )


def _transform(features, weight, bias):
    grid = (pl.cdiv(N_SITES, BN),)
    return pl.pallas_call(
        _mm_body,
        grid=grid,
        in_specs=[
            pl.BlockSpec((BN, N_IN), lambda i: (i, 0)),
            pl.BlockSpec((FV, N_IN, N_OUT), lambda i: (0, 0, 0)),
            pl.BlockSpec((1, N_OUT), lambda i: (0, 0)),
        ],
        out_specs=pl.BlockSpec((FV, BN, N_OUT), lambda i: (0, i, 0)),
        out_shape=jax.ShapeDtypeStruct((FV, N_SITES, N_OUT), jnp.float32),
    )(features, weight, bias.reshape(1, N_OUT))


def _work(t_hbm, idx_hbm, out_hbm, idx_v, acc_v, sem, base, chunk, sub):
    pltpu.sync_copy(
        idx_hbm.at[pl.ds(base * FV, FV * chunk)],
        idx_v.at[pl.ds(0, FV * chunk)],
    )
    for i in range(chunk // sub):
        off = base + i * sub
        acc = acc_v.at[pl.ds(0, sub)]
        pltpu.async_copy(
            t_hbm.at[idx_v.at[pl.ds(i * sub, sub)]], acc, sem
        ).wait()
        for f in range(1, FV):
            pltpu.async_copy(
                t_hbm.at[idx_v.at[pl.ds(f * chunk + i * sub, sub)]],
                acc,
                sem,
                add=True,
            ).wait()
        pltpu.sync_copy(acc, out_hbm.at[pl.ds(off, sub)])


def _sc_body(t_hbm, idx_hbm, out_hbm, idx_v, acc_v, sem):
    c = lax.axis_index("c")
    s = lax.axis_index("s")
    wid = s * 2 + c
    base = wid * CHUNK

    @pl.when(wid < NW - 1)
    def _full():
        _work(t_hbm, idx_hbm, out_hbm, idx_v, acc_v, sem, base, CHUNK, SUB)

    @pl.when(wid == NW - 1)
    def _last():
        _work(t_hbm, idx_hbm, out_hbm, idx_v, acc_v, sem, base, CHUNK_L, SUB_L)


_gather_sum = functools.partial(
    pl.kernel,
    out_type=jax.ShapeDtypeStruct((N_SITES, N_OUT), jnp.float32),
    mesh=plsc.VectorSubcoreMesh(core_axis_name="c", subcore_axis_name="s"),
    scratch_types=[
        pltpu.VMEM((FV * CHUNK,), jnp.int32),
        pltpu.VMEM((SUB, N_OUT), jnp.float32),
        pltpu.SemaphoreType.DMA,
    ],
)(_sc_body)


@jax.jit
def kernel(features, neighbor_idx, weight, bias):
    t = _transform(features, weight, bias)
    t_flat = t.reshape(FV * N_SITES, N_OUT)
    idx_t = (
        neighbor_idx.T
        + (jnp.arange(FV, dtype=jnp.int32) * N_SITES)[:, None]
    )
    full = (
        idx_t[:, : (NW - 1) * CHUNK]
        .reshape(FV, NW - 1, CHUNK)
        .transpose(1, 0, 2)
        .reshape(-1)
    )
    last = idx_t[:, (NW - 1) * CHUNK:].reshape(-1)
    return _gather_sum(t_flat, jnp.concatenate([full, last]))
